# Initial kernel scaffold; baseline (speedup 1.0000x reference)
import jax
import jax.numpy as jnp
from jax.experimental import pallas as pl


def kernel(fmel, fblood, skincolor):
    raise NotImplementedError("write your pallas kernel here")


# speedup vs baseline: 31.5126x; 31.5126x over previous
import functools

import jax
import jax.numpy as jnp
from jax import lax
from jax.experimental import pallas as pl
from jax.experimental.pallas import tpu as pltpu
from jax.experimental.pallas import tpu_sc as plsc

NC = 2
NS = 16
NW = NC * NS

B, H, W = 4, 512, 512
NPIX = B * H * W
CH = 33
D = 144
P = 128
CHUNKS = NPIX // P
CHUNKS_PER_W = CHUNKS // NW


def _sc_body(t4_hbm, fm_hbm, fb_hbm, out_hbm,
             fm_v, fb_v, idx_v, w00_v, w01_v, w10_v, w11_v, g_v, out_v, sem):
    cidx = lax.axis_index("c")
    sidx = lax.axis_index("s")
    wid = sidx * NC + cidx

    def chunk_body(t, carry):
        cid = wid * CHUNKS_PER_W + t
        base = cid * P
        pltpu.sync_copy(fm_hbm.at[pl.ds(base, P)], fm_v)
        pltpu.sync_copy(fb_hbm.at[pl.ds(base, P)], fb_v)

        def grp(j, c2):
            p0 = pl.multiple_of(j * 16, 16)
            x = fb_v[pl.ds(p0, 16)]
            y = fm_v[pl.ds(p0, 16)]
            ix = jnp.clip(x * 128.0 + 127.5, 0.0, 255.0)
            iy = jnp.clip(y * 128.0 + 127.5, 0.0, 255.0)
            ix0 = jnp.minimum(ix.astype(jnp.int32), 254)
            iy0 = jnp.minimum(iy.astype(jnp.int32), 254)
            wx1 = ix - ix0.astype(jnp.float32)
            wy1 = iy - iy0.astype(jnp.float32)
            wx0 = 1.0 - wx1
            wy0 = 1.0 - wy1
            idx_v[pl.ds(p0, 16)] = iy0 * 256 + ix0
            w00_v[pl.ds(p0, 16)] = wy0 * wx0
            w01_v[pl.ds(p0, 16)] = wy0 * wx1
            w10_v[pl.ds(p0, 16)] = wy1 * wx0
            w11_v[pl.ds(p0, 16)] = wy1 * wx1
            return c2

        lax.fori_loop(0, P // 16, grp, 0)

        pltpu.async_copy(t4_hbm.at[idx_v], g_v, sem).wait()

        def grp2(j, c2):
            p0 = pl.multiple_of(j * 16, 16)
            rows = lax.iota(jnp.int32, 16) + p0
            w00 = w00_v[pl.ds(p0, 16)]
            w01 = w01_v[pl.ds(p0, 16)]
            w10 = w10_v[pl.ds(p0, 16)]
            w11 = w11_v[pl.ds(p0, 16)]
            for ch in range(CH):
                v00 = plsc.load_gather(g_v, [rows, jnp.full((16,), ch, jnp.int32)])
                v01 = plsc.load_gather(g_v, [rows, jnp.full((16,), ch + 33, jnp.int32)])
                v10 = plsc.load_gather(g_v, [rows, jnp.full((16,), ch + 66, jnp.int32)])
                v11 = plsc.load_gather(g_v, [rows, jnp.full((16,), ch + 99, jnp.int32)])
                out_v[0, ch, 0, pl.ds(p0, 16)] = (w00 * v00 + w01 * v01) + (w10 * v10 + w11 * v11)
            return c2

        lax.fori_loop(0, P // 16, grp2, 0)

        b = cid // (H * (W // P))
        r = cid % (H * (W // P))
        i = r // (W // P)
        j0 = (r % (W // P)) * P
        pltpu.sync_copy(out_v, out_hbm.at[pl.ds(b, 1), :, pl.ds(i, 1), pl.ds(j0, P)])
        return carry

    lax.fori_loop(0, CHUNKS_PER_W, chunk_body, 0)


@functools.partial(
    pl.kernel,
    mesh=plsc.VectorSubcoreMesh(core_axis_name="c", subcore_axis_name="s"),
    out_type=jax.ShapeDtypeStruct((B, CH, H, W), jnp.float32),
    compiler_params=pltpu.CompilerParams(
        use_tc_tiling_on_sc=False, needs_layout_passes=False
    ),
    scratch_types=[
        pltpu.VMEM((P,), jnp.float32),
        pltpu.VMEM((P,), jnp.float32),
        pltpu.VMEM((P,), jnp.int32),
        pltpu.VMEM((P,), jnp.float32),
        pltpu.VMEM((P,), jnp.float32),
        pltpu.VMEM((P,), jnp.float32),
        pltpu.VMEM((P,), jnp.float32),
        pltpu.VMEM((P, D), jnp.float32),
        pltpu.VMEM((1, CH, 1, P), jnp.float32),
        pltpu.SemaphoreType.DMA,
    ],
)
def _sc_kernel(t4_hbm, fm_hbm, fb_hbm, out_hbm, *rest):
    _sc_body(t4_hbm, fm_hbm, fb_hbm, out_hbm, *rest)


def kernel(fmel, fblood, skincolor):
    sc = skincolor[0]
    scx = jnp.concatenate([sc[:, 1:], sc[:, 255:]], axis=1)
    scy = jnp.concatenate([sc[1:], sc[255:]], axis=0)
    scxy = jnp.concatenate([scy[:, 1:], scy[:, 255:]], axis=1)
    pad = jnp.zeros((256, 256, D - 4 * CH), jnp.float32)
    t4 = jnp.concatenate([sc, scx, scy, scxy, pad], axis=-1).reshape(256 * 256, D)
    fm_flat = fmel.reshape(NPIX)
    fb_flat = fblood.reshape(NPIX)
    return _sc_kernel(t4, fm_flat, fb_flat)


# speedup vs baseline: 44.3652x; 1.4079x over previous
import functools

import jax
import jax.numpy as jnp
from jax import lax
from jax.experimental import pallas as pl
from jax.experimental.pallas import tpu as pltpu
from jax.experimental.pallas import tpu_sc as plsc

NC = 2
NS = 16
NW = NC * NS

B, H, W = 4, 512, 512
NPIX = B * H * W
CH = 33
D = 144
P = 256
G = 128
NG = P // G
CHUNKS = NPIX // P
CPW = CHUNKS // NW
ROW_CHUNKS = W // P


def _sc_body(t4_hbm, fm_hbm, fb_hbm, out_hbm,
             fm_v, fb_v, idx_v, w00_v, w01_v, w10_v, w11_v, g_v, out_v,
             sem_in0, sem_in1, sem_g0, sem_g1, sem_o0, sem_o1):
    sem_in = (sem_in0, sem_in1)
    sem_g = (sem_g0, sem_g1)
    sem_o = (sem_o0, sem_o1)
    wid = lax.axis_index("s") * NC + lax.axis_index("c")
    c0 = wid * CPW

    def out_dst(cid):
        b = cid // (H * ROW_CHUNKS)
        r = cid % (H * ROW_CHUNKS)
        i = r // ROW_CHUNKS
        j0 = (r % ROW_CHUNKS) * P
        return out_hbm.at[pl.ds(b, 1), :, pl.ds(i, 1), pl.ds(j0, P)]

    def issue_in(slot, cid):
        base = cid * P
        pltpu.async_copy(fm_hbm.at[pl.ds(base, P)], fm_v.at[slot], sem_in[slot])
        pltpu.async_copy(fb_hbm.at[pl.ds(base, P)], fb_v.at[slot], sem_in[slot])

    def do_mid(slot, cid):
        base = cid * P
        pltpu.make_async_copy(fm_hbm.at[pl.ds(base, P)], fm_v.at[slot], sem_in[slot]).wait()
        pltpu.make_async_copy(fb_hbm.at[pl.ds(base, P)], fb_v.at[slot], sem_in[slot]).wait()
        for k in range(NG):
            def grp(j, c2, k=k):
                p0 = pl.multiple_of(k * G + j * 16, 16)
                x = fb_v[slot, pl.ds(p0, 16)]
                y = fm_v[slot, pl.ds(p0, 16)]
                ix = jnp.clip(x * 128.0 + 127.5, 0.0, 255.0)
                iy = jnp.clip(y * 128.0 + 127.5, 0.0, 255.0)
                ix0 = jnp.minimum(ix.astype(jnp.int32), 254)
                iy0 = jnp.minimum(iy.astype(jnp.int32), 254)
                wx1 = ix - ix0.astype(jnp.float32)
                wy1 = iy - iy0.astype(jnp.float32)
                wx0 = 1.0 - wx1
                wy0 = 1.0 - wy1
                idx_v[slot, k, pl.ds(j * 16, 16)] = iy0 * 256 + ix0
                w00_v[slot, pl.ds(p0, 16)] = wy0 * wx0
                w01_v[slot, pl.ds(p0, 16)] = wy0 * wx1
                w10_v[slot, pl.ds(p0, 16)] = wy1 * wx0
                w11_v[slot, pl.ds(p0, 16)] = wy1 * wx1
                return c2

            lax.fori_loop(0, G // 16, grp, 0)
            pltpu.async_copy(t4_hbm.at[idx_v.at[slot, k]],
                             g_v.at[slot, pl.ds(k * G, G)], sem_g[slot])

    def do_out(slot, cid, s):
        pltpu.make_async_copy(t4_hbm.at[pl.ds(0, P)], g_v.at[slot], sem_g[slot]).wait()
        dst = out_dst(cid)

        @pl.when(s >= 2)
        def _():
            pltpu.make_async_copy(out_v.at[slot], dst, sem_o[slot]).wait()

        slot_i = jnp.full((16,), slot, jnp.int32)

        def grp2(j, c2):
            p0 = pl.multiple_of(j * 16, 16)
            rows = lax.iota(jnp.int32, 16) + p0
            w00 = w00_v[slot, pl.ds(p0, 16)]
            w01 = w01_v[slot, pl.ds(p0, 16)]
            w10 = w10_v[slot, pl.ds(p0, 16)]
            w11 = w11_v[slot, pl.ds(p0, 16)]
            for ch in range(CH):
                v00 = plsc.load_gather(g_v, [slot_i, rows, jnp.full((16,), ch, jnp.int32)])
                v01 = plsc.load_gather(g_v, [slot_i, rows, jnp.full((16,), ch + 33, jnp.int32)])
                v10 = plsc.load_gather(g_v, [slot_i, rows, jnp.full((16,), ch + 66, jnp.int32)])
                v11 = plsc.load_gather(g_v, [slot_i, rows, jnp.full((16,), ch + 99, jnp.int32)])
                out_v[slot, 0, ch, 0, pl.ds(p0, 16)] = (
                    (w00 * v00 + w01 * v01) + (w10 * v10 + w11 * v11))
            return c2

        lax.fori_loop(0, P // 16, grp2, 0)
        pltpu.async_copy(out_v.at[slot], dst, sem_o[slot])

    issue_in(0, c0)
    issue_in(1, c0 + 1)
    do_mid(0, c0)

    def iter_body(u, carry):
        for h in range(2):
            s = 2 * u + h
            cid = c0 + s

            @pl.when(s + 1 < CPW)
            def _(h=h, s=s, cid=cid):
                do_mid(1 - h, cid + 1)

            @pl.when(s + 2 < CPW)
            def _(h=h, s=s, cid=cid):
                issue_in(h, cid + 2)

            do_out(h, cid, s)
        return carry

    lax.fori_loop(0, CPW // 2, iter_body, 0)

    for slot in range(2):
        cid = c0 + CPW - 2 + slot
        pltpu.make_async_copy(out_v.at[slot], out_dst(cid), sem_o[slot]).wait()


@functools.partial(
    pl.kernel,
    mesh=plsc.VectorSubcoreMesh(core_axis_name="c", subcore_axis_name="s"),
    out_type=jax.ShapeDtypeStruct((B, CH, H, W), jnp.float32),
    compiler_params=pltpu.CompilerParams(
        use_tc_tiling_on_sc=False, needs_layout_passes=False
    ),
    scratch_types=[
        pltpu.VMEM((2, P), jnp.float32),
        pltpu.VMEM((2, P), jnp.float32),
        pltpu.VMEM((2, NG, G), jnp.int32),
        pltpu.VMEM((2, P), jnp.float32),
        pltpu.VMEM((2, P), jnp.float32),
        pltpu.VMEM((2, P), jnp.float32),
        pltpu.VMEM((2, P), jnp.float32),
        pltpu.VMEM((2, P, D), jnp.float32),
        pltpu.VMEM((2, 1, CH, 1, P), jnp.float32),
        pltpu.SemaphoreType.DMA,
        pltpu.SemaphoreType.DMA,
        pltpu.SemaphoreType.DMA,
        pltpu.SemaphoreType.DMA,
        pltpu.SemaphoreType.DMA,
        pltpu.SemaphoreType.DMA,
    ],
)
def _sc_kernel(t4_hbm, fm_hbm, fb_hbm, out_hbm, *rest):
    _sc_body(t4_hbm, fm_hbm, fb_hbm, out_hbm, *rest)


def kernel(fmel, fblood, skincolor):
    sc = skincolor[0]
    scx = jnp.concatenate([sc[:, 1:], sc[:, 255:]], axis=1)
    scy = jnp.concatenate([sc[1:], sc[255:]], axis=0)
    scxy = jnp.concatenate([scy[:, 1:], scy[:, 255:]], axis=1)
    pad = jnp.zeros((256, 256, D - 4 * CH), jnp.float32)
    t4 = jnp.concatenate([sc, scx, scy, scxy, pad], axis=-1).reshape(256 * 256, D)
    fm_flat = fmel.reshape(NPIX)
    fb_flat = fblood.reshape(NPIX)
    return _sc_kernel(t4, fm_flat, fb_flat)


# speedup vs baseline: 54.4068x; 1.2263x over previous
import functools

import jax
import jax.numpy as jnp
from jax import lax
from jax.experimental import pallas as pl
from jax.experimental.pallas import tpu as pltpu
from jax.experimental.pallas import tpu_sc as plsc

NC = 2
NS = 16
NW = NC * NS

B, H, W = 4, 512, 512
NPIX = B * H * W
CH = 33
D = 144
P = 256
G = 128
NG = P // G
CHUNKS = NPIX // P
CPW = CHUNKS // NW
ROW_CHUNKS = W // P


def _sc_body(t4_hbm, fm_hbm, fb_hbm, out_hbm,
             fm_v, fb_v, idx_v, w00_v, w01_v, w10_v, w11_v, g_v, out_v,
             sem_in0, sem_in1, sem_g0, sem_g1, sem_o0, sem_o1):
    sem_in = (sem_in0, sem_in1)
    sem_g = (sem_g0, sem_g1)
    sem_o = (sem_o0, sem_o1)
    wid = lax.axis_index("s") * NC + lax.axis_index("c")
    c0 = wid * CPW

    def out_dst(cid):
        b = cid // (H * ROW_CHUNKS)
        r = cid % (H * ROW_CHUNKS)
        i = r // ROW_CHUNKS
        j0 = (r % ROW_CHUNKS) * P
        return out_hbm.at[pl.ds(b, 1), :, pl.ds(i, 1), pl.ds(j0, P)]

    def issue_in(slot, cid):
        base = cid * P
        pltpu.async_copy(fm_hbm.at[pl.ds(base, P)], fm_v.at[slot], sem_in[slot])
        pltpu.async_copy(fb_hbm.at[pl.ds(base, P)], fb_v.at[slot], sem_in[slot])

    def do_mid(slot, cid):
        base = cid * P
        pltpu.make_async_copy(fm_hbm.at[pl.ds(base, P)], fm_v.at[slot], sem_in[slot]).wait()
        pltpu.make_async_copy(fb_hbm.at[pl.ds(base, P)], fb_v.at[slot], sem_in[slot]).wait()
        for k in range(NG):
            def grp(j, c2, k=k):
                p0 = pl.multiple_of(k * G + j * 16, 16)
                x = fb_v[slot, pl.ds(p0, 16)]
                y = fm_v[slot, pl.ds(p0, 16)]
                ix = jnp.clip(x * 128.0 + 127.5, 0.0, 255.0)
                iy = jnp.clip(y * 128.0 + 127.5, 0.0, 255.0)
                ix0 = jnp.minimum(ix.astype(jnp.int32), 254)
                iy0 = jnp.minimum(iy.astype(jnp.int32), 254)
                wx1 = ix - ix0.astype(jnp.float32)
                wy1 = iy - iy0.astype(jnp.float32)
                wx0 = 1.0 - wx1
                wy0 = 1.0 - wy1
                idx_v[slot, k, pl.ds(j * 16, 16)] = iy0 * 256 + ix0
                w00_v[slot, pl.ds(p0, 16)] = wy0 * wx0
                w01_v[slot, pl.ds(p0, 16)] = wy0 * wx1
                w10_v[slot, pl.ds(p0, 16)] = wy1 * wx0
                w11_v[slot, pl.ds(p0, 16)] = wy1 * wx1
                return c2

            lax.fori_loop(0, G // 16, grp, 0)
            pltpu.async_copy(t4_hbm.at[idx_v.at[slot, k]],
                             g_v.at[slot, pl.ds(k * G, G)], sem_g[slot])

    def do_out(slot, cid, s):
        pltpu.make_async_copy(t4_hbm.at[pl.ds(0, P)], g_v.at[slot], sem_g[slot]).wait()
        dst = out_dst(cid)

        @pl.when(s >= 2)
        def _():
            pltpu.make_async_copy(out_v.at[slot], dst, sem_o[slot]).wait()

        slot_i = jnp.full((16,), slot, jnp.int32)

        def grp2(j, c2):
            p0 = pl.multiple_of(j * 16, 16)
            rows = lax.iota(jnp.int32, 16) + p0
            w00 = w00_v[slot, pl.ds(p0, 16)]
            w01 = w01_v[slot, pl.ds(p0, 16)]
            w10 = w10_v[slot, pl.ds(p0, 16)]
            w11 = w11_v[slot, pl.ds(p0, 16)]

            @plsc.parallel_loop(0, CH, unroll=4)
            def chloop(ch):
                cols = jnp.full((16,), ch, jnp.int32)
                v00 = plsc.load_gather(g_v, [slot_i, rows, cols])
                v01 = plsc.load_gather(g_v, [slot_i, rows, cols + 33])
                v10 = plsc.load_gather(g_v, [slot_i, rows, cols + 66])
                v11 = plsc.load_gather(g_v, [slot_i, rows, cols + 99])
                out_v[slot, 0, ch, 0, pl.ds(p0, 16)] = (
                    (w00 * v00 + w01 * v01) + (w10 * v10 + w11 * v11))

            return c2

        lax.fori_loop(0, P // 16, grp2, 0)
        pltpu.async_copy(out_v.at[slot], dst, sem_o[slot])

    issue_in(0, c0)
    issue_in(1, c0 + 1)
    do_mid(0, c0)

    def iter_body(u, carry):
        for h in range(2):
            s = 2 * u + h
            cid = c0 + s

            @pl.when(s + 1 < CPW)
            def _(h=h, s=s, cid=cid):
                do_mid(1 - h, cid + 1)

            @pl.when(s + 2 < CPW)
            def _(h=h, s=s, cid=cid):
                issue_in(h, cid + 2)

            do_out(h, cid, s)
        return carry

    lax.fori_loop(0, CPW // 2, iter_body, 0)

    for slot in range(2):
        cid = c0 + CPW - 2 + slot
        pltpu.make_async_copy(out_v.at[slot], out_dst(cid), sem_o[slot]).wait()


@functools.partial(
    pl.kernel,
    mesh=plsc.VectorSubcoreMesh(core_axis_name="c", subcore_axis_name="s"),
    out_type=jax.ShapeDtypeStruct((B, CH, H, W), jnp.float32),
    compiler_params=pltpu.CompilerParams(
        use_tc_tiling_on_sc=False, needs_layout_passes=False
    ),
    scratch_types=[
        pltpu.VMEM((2, P), jnp.float32),
        pltpu.VMEM((2, P), jnp.float32),
        pltpu.VMEM((2, NG, G), jnp.int32),
        pltpu.VMEM((2, P), jnp.float32),
        pltpu.VMEM((2, P), jnp.float32),
        pltpu.VMEM((2, P), jnp.float32),
        pltpu.VMEM((2, P), jnp.float32),
        pltpu.VMEM((2, P, D), jnp.float32),
        pltpu.VMEM((2, 1, CH, 1, P), jnp.float32),
        pltpu.SemaphoreType.DMA,
        pltpu.SemaphoreType.DMA,
        pltpu.SemaphoreType.DMA,
        pltpu.SemaphoreType.DMA,
        pltpu.SemaphoreType.DMA,
        pltpu.SemaphoreType.DMA,
    ],
)
def _sc_kernel(t4_hbm, fm_hbm, fb_hbm, out_hbm, *rest):
    _sc_body(t4_hbm, fm_hbm, fb_hbm, out_hbm, *rest)


def kernel(fmel, fblood, skincolor):
    sc = skincolor[0]
    scx = jnp.concatenate([sc[:, 1:], sc[:, 255:]], axis=1)
    scy = jnp.concatenate([sc[1:], sc[255:]], axis=0)
    scxy = jnp.concatenate([scy[:, 1:], scy[:, 255:]], axis=1)
    pad = jnp.zeros((256, 256, D - 4 * CH), jnp.float32)
    t4 = jnp.concatenate([sc, scx, scy, scxy, pad], axis=-1).reshape(256 * 256, D)
    fm_flat = fmel.reshape(NPIX)
    fb_flat = fblood.reshape(NPIX)
    return _sc_kernel(t4, fm_flat, fb_flat)


# speedup vs baseline: 72.6666x; 1.3356x over previous
import functools

import jax
import jax.numpy as jnp
from jax import lax
from jax.experimental import pallas as pl
from jax.experimental.pallas import tpu as pltpu
from jax.experimental.pallas import tpu_sc as plsc

NC = 2
NS = 16
NW = NC * NS

B, H, W = 4, 512, 512
NPIX = B * H * W
CH = 33
D = 144
P = 256
G = 128
NG = P // G
CHUNKS = NPIX // P
CPW = CHUNKS // NW
ROW_CHUNKS = W // P


def _sc_body(t4_hbm, fm_hbm, fb_hbm, out_hbm,
             fm_v, fb_v, idx_v, w00_v, w01_v, w10_v, w11_v, g_v, out_v,
             sem_in0, sem_in1, sem_g0, sem_g1, sem_o0, sem_o1):
    sem_in = (sem_in0, sem_in1)
    sem_g = (sem_g0, sem_g1)
    sem_o = (sem_o0, sem_o1)
    wid = lax.axis_index("s") * NC + lax.axis_index("c")
    c0 = wid * CPW

    def out_dst(cid):
        b = cid // (H * ROW_CHUNKS)
        r = cid % (H * ROW_CHUNKS)
        i = r // ROW_CHUNKS
        j0 = (r % ROW_CHUNKS) * P
        return out_hbm.at[pl.ds(b, 1), :, pl.ds(i, 1), pl.ds(j0, P)]

    def issue_in(slot, cid):
        base = cid * P
        pltpu.async_copy(fm_hbm.at[pl.ds(base, P)], fm_v.at[slot], sem_in[slot])
        pltpu.async_copy(fb_hbm.at[pl.ds(base, P)], fb_v.at[slot], sem_in[slot])

    def do_mid(slot, cid):
        base = cid * P
        pltpu.make_async_copy(fm_hbm.at[pl.ds(base, P)], fm_v.at[slot], sem_in[slot]).wait()
        pltpu.make_async_copy(fb_hbm.at[pl.ds(base, P)], fb_v.at[slot], sem_in[slot]).wait()
        for k in range(NG):
            @plsc.parallel_loop(0, G // 16, unroll=2)
            def grp(j, k=k):
                p0 = pl.multiple_of(k * G + j * 16, 16)
                x = fb_v[slot, pl.ds(p0, 16)]
                y = fm_v[slot, pl.ds(p0, 16)]
                ix = jnp.clip(x * 128.0 + 127.5, 0.0, 255.0)
                iy = jnp.clip(y * 128.0 + 127.5, 0.0, 255.0)
                ix0 = jnp.minimum(ix.astype(jnp.int32), 254)
                iy0 = jnp.minimum(iy.astype(jnp.int32), 254)
                wx1 = ix - ix0.astype(jnp.float32)
                wy1 = iy - iy0.astype(jnp.float32)
                wx0 = 1.0 - wx1
                wy0 = 1.0 - wy1
                idx_v[slot, k, pl.ds(j * 16, 16)] = iy0 * 256 + ix0
                w00_v[slot, pl.ds(p0, 16)] = wy0 * wx0
                w01_v[slot, pl.ds(p0, 16)] = wy0 * wx1
                w10_v[slot, pl.ds(p0, 16)] = wy1 * wx0
                w11_v[slot, pl.ds(p0, 16)] = wy1 * wx1

            pltpu.async_copy(t4_hbm.at[idx_v.at[slot, k]],
                             g_v.at[slot, pl.ds(k * G, G)], sem_g[slot])

    def do_out(slot, cid, s):
        pltpu.make_async_copy(t4_hbm.at[pl.ds(0, P)], g_v.at[slot], sem_g[slot]).wait()
        dst = out_dst(cid)

        @pl.when(s >= 2)
        def _():
            pltpu.make_async_copy(out_v.at[slot], dst, sem_o[slot]).wait()

        zero16 = jnp.zeros((16,), jnp.int32)
        iota_d = lax.iota(jnp.int32, 16) * D

        def grp2(j, c2):
            p0 = pl.multiple_of(j * 16, 16)
            w00 = w00_v[slot, pl.ds(p0, 16)]
            w01 = w01_v[slot, pl.ds(p0, 16)]
            w10 = w10_v[slot, pl.ds(p0, 16)]
            w11 = w11_v[slot, pl.ds(p0, 16)]
            a00 = iota_d + (slot * P * D + p0 * D)
            carry0 = (a00, a00 + 33, a00 + 66, a00 + 99)

            @plsc.parallel_loop(0, CH, unroll=4, carry=carry0)
            def chloop(ch, addrs):
                a0, a1, a2, a3 = addrs
                v00 = plsc.load_gather(g_v, [zero16, zero16, a0])
                v01 = plsc.load_gather(g_v, [zero16, zero16, a1])
                v10 = plsc.load_gather(g_v, [zero16, zero16, a2])
                v11 = plsc.load_gather(g_v, [zero16, zero16, a3])
                out_v[slot, 0, ch, 0, pl.ds(p0, 16)] = (
                    (w00 * v00 + w01 * v01) + (w10 * v10 + w11 * v11))
                return (a0 + 1, a1 + 1, a2 + 1, a3 + 1)

            return c2

        lax.fori_loop(0, P // 16, grp2, 0)
        pltpu.async_copy(out_v.at[slot], dst, sem_o[slot])

    issue_in(0, c0)
    issue_in(1, c0 + 1)
    do_mid(0, c0)

    def iter_body(u, carry):
        for h in range(2):
            s = 2 * u + h
            cid = c0 + s

            @pl.when(s + 1 < CPW)
            def _(h=h, s=s, cid=cid):
                do_mid(1 - h, cid + 1)

            @pl.when(s + 2 < CPW)
            def _(h=h, s=s, cid=cid):
                issue_in(h, cid + 2)

            do_out(h, cid, s)
        return carry

    lax.fori_loop(0, CPW // 2, iter_body, 0)

    for slot in range(2):
        cid = c0 + CPW - 2 + slot
        pltpu.make_async_copy(out_v.at[slot], out_dst(cid), sem_o[slot]).wait()


@functools.partial(
    pl.kernel,
    mesh=plsc.VectorSubcoreMesh(core_axis_name="c", subcore_axis_name="s"),
    out_type=jax.ShapeDtypeStruct((B, CH, H, W), jnp.float32),
    compiler_params=pltpu.CompilerParams(
        use_tc_tiling_on_sc=False, needs_layout_passes=False
    ),
    scratch_types=[
        pltpu.VMEM((2, P), jnp.float32),
        pltpu.VMEM((2, P), jnp.float32),
        pltpu.VMEM((2, NG, G), jnp.int32),
        pltpu.VMEM((2, P), jnp.float32),
        pltpu.VMEM((2, P), jnp.float32),
        pltpu.VMEM((2, P), jnp.float32),
        pltpu.VMEM((2, P), jnp.float32),
        pltpu.VMEM((2, P, D), jnp.float32),
        pltpu.VMEM((2, 1, CH, 1, P), jnp.float32),
        pltpu.SemaphoreType.DMA,
        pltpu.SemaphoreType.DMA,
        pltpu.SemaphoreType.DMA,
        pltpu.SemaphoreType.DMA,
        pltpu.SemaphoreType.DMA,
        pltpu.SemaphoreType.DMA,
    ],
)
def _sc_kernel(t4_hbm, fm_hbm, fb_hbm, out_hbm, *rest):
    _sc_body(t4_hbm, fm_hbm, fb_hbm, out_hbm, *rest)


def kernel(fmel, fblood, skincolor):
    sc = skincolor[0]
    scx = jnp.concatenate([sc[:, 1:], sc[:, 255:]], axis=1)
    scy = jnp.concatenate([sc[1:], sc[255:]], axis=0)
    scxy = jnp.concatenate([scy[:, 1:], scy[:, 255:]], axis=1)
    pad = jnp.zeros((256, 256, D - 4 * CH), jnp.float32)
    t4 = jnp.concatenate([sc, scx, scy, scxy, pad], axis=-1).reshape(256 * 256, D)
    fm_flat = fmel.reshape(NPIX)
    fb_flat = fblood.reshape(NPIX)
    return _sc_kernel(t4, fm_flat, fb_flat)
